# Initial kernel scaffold; baseline (speedup 1.0000x reference)
import jax
import jax.numpy as jnp
from jax.experimental import pallas as pl


def kernel(x, edge_index, W0, b0, W1, b1, W2, b2, W3, b3):
    raise NotImplementedError("write your pallas kernel here")


# speedup vs baseline: 2.0862x; 2.0862x over previous
import functools

import jax
import jax.numpy as jnp
from jax import lax
from jax.experimental import pallas as pl
from jax.experimental.pallas import tpu as pltpu

N = 10000
E = 160000
DIN, DH, DOUT = 256, 512, 128
M_BLK = 1000
GRID_M = N // M_BLK
NCHUNK = DH // 128


def _mm0_body(x_ref, w_ref, g_ref):
    g_ref[...] = jnp.dot(x_ref[...], w_ref[...],
                         preferred_element_type=jnp.float32)


def _tc_mm0(x, w0):
    return pl.pallas_call(
        _mm0_body,
        grid=(GRID_M,),
        in_specs=[
            pl.BlockSpec((M_BLK, DIN), lambda m: (m, 0)),
            pl.BlockSpec((DIN, DH), lambda m: (0, 0)),
        ],
        out_specs=pl.BlockSpec((M_BLK, DH), lambda m: (m, 0)),
        out_shape=jax.ShapeDtypeStruct((N, DH), jnp.float32),
    )(x, w0)


def _deg_terms(cnt_ref):
    cnt = cnt_ref[...]
    deg = cnt[0] + cnt[1] + 1.0
    dis = lax.rsqrt(deg)
    inv = 1.0 / deg
    return dis, inv


def _scale_body(g_ref, cnt_ref, gs_ref):
    dis, _ = _deg_terms(cnt_ref)
    gd = g_ref[...] * dis
    for c in range(NCHUNK):
        gs_ref[c] = gd[:, 128 * c:128 * (c + 1)]


def _tc_scale0(g0, cnt2):
    return pl.pallas_call(
        _scale_body,
        grid=(GRID_M,),
        in_specs=[
            pl.BlockSpec((M_BLK, DH), lambda m: (m, 0)),
            pl.BlockSpec((2, M_BLK, 1), lambda m: (0, m, 0)),
        ],
        out_specs=pl.BlockSpec((NCHUNK, M_BLK, 128), lambda m: (0, m, 0)),
        out_shape=jax.ShapeDtypeStruct((NCHUNK, N, 128), jnp.float32),
    )(g0, cnt2)


def _mid_body(dout, s_ref, gp_ref, cnt_ref, b_ref, w_ref, g_ref, gs_ref):
    dis, inv = _deg_terms(cnt_ref)
    h = jnp.maximum(dis * s_ref[...] + inv * gp_ref[...] + b_ref[...], 0.0)
    g = jnp.dot(h, w_ref[...], preferred_element_type=jnp.float32)
    g_ref[...] = g
    gd = g * dis
    nchunk = dout // 128
    for c in range(nchunk):
        gs_ref[c] = gd[:, 128 * c:128 * (c + 1)]


def _tc_mid(s, g_prev, cnt2, b, w):
    din, dout = w.shape
    nchunk = dout // 128
    return pl.pallas_call(
        functools.partial(_mid_body, dout),
        grid=(GRID_M,),
        in_specs=[
            pl.BlockSpec((M_BLK, din), lambda m: (m, 0)),
            pl.BlockSpec((M_BLK, din), lambda m: (m, 0)),
            pl.BlockSpec((2, M_BLK, 1), lambda m: (0, m, 0)),
            pl.BlockSpec((1, din), lambda m: (0, 0)),
            pl.BlockSpec((din, dout), lambda m: (0, 0)),
        ],
        out_specs=[
            pl.BlockSpec((M_BLK, dout), lambda m: (m, 0)),
            pl.BlockSpec((nchunk, M_BLK, 128), lambda m: (0, m, 0)),
        ],
        out_shape=[
            jax.ShapeDtypeStruct((N, dout), jnp.float32),
            jax.ShapeDtypeStruct((nchunk, N, 128), jnp.float32),
        ],
    )(s, g_prev, cnt2, b, w)


def _final_body(s_ref, g_ref, cnt_ref, b_ref, o_ref):
    dis, inv = _deg_terms(cnt_ref)
    s = s_ref[0] + s_ref[1]
    pre = dis * s + inv * g_ref[...] + b_ref[...]
    m = jnp.max(pre, axis=1, keepdims=True)
    y = pre - m
    lse = jnp.log(jnp.sum(jnp.exp(y), axis=1, keepdims=True))
    o_ref[...] = y - lse


def _tc_final(s3p, g3, cnt2, b3):
    return pl.pallas_call(
        _final_body,
        grid=(GRID_M,),
        in_specs=[
            pl.BlockSpec((2, M_BLK, DOUT), lambda m: (0, m, 0)),
            pl.BlockSpec((M_BLK, DOUT), lambda m: (m, 0)),
            pl.BlockSpec((2, M_BLK, 1), lambda m: (0, m, 0)),
            pl.BlockSpec((1, DOUT), lambda m: (0, 0)),
        ],
        out_specs=pl.BlockSpec((M_BLK, DOUT), lambda m: (m, 0)),
        out_shape=jax.ShapeDtypeStruct((N, DOUT), jnp.float32),
    )(s3p, g3, cnt2, b3)


def kernel(x, edge_index, W0, b0, W1, b1, W2, b2, W3, b3):
    src = edge_index[0]
    dst = edge_index[1]

    cnt = jnp.zeros((N,), jnp.float32).at[dst].add(1.0)
    cnt2 = jnp.stack([cnt, jnp.zeros_like(cnt)]).reshape(2, N, 1)

    def prop(gs_chunked):
        nchunk = gs_chunked.shape[0]
        gs = jnp.transpose(gs_chunked, (1, 0, 2)).reshape(N, nchunk * 128)
        return jnp.zeros((N, nchunk * 128), jnp.float32).at[dst].add(gs[src])

    g0 = _tc_mm0(x, W0)
    gs0 = _tc_scale0(g0, cnt2)
    s0 = prop(gs0)
    g1, gs1 = _tc_mid(s0, g0, cnt2, b0.reshape(1, DH), W1)
    s1 = prop(gs1)
    g2, gs2 = _tc_mid(s1, g1, cnt2, b1.reshape(1, DH), W2)
    s2 = prop(gs2)
    g3, gs3 = _tc_mid(s2, g2, cnt2, b2.reshape(1, DH), W3)
    s3 = prop(gs3)
    s3p = jnp.stack([s3, jnp.zeros_like(s3)])
    return _tc_final(s3p, g3, cnt2, b3.reshape(1, DOUT))


# speedup vs baseline: 5.6072x; 2.6877x over previous
import functools

import jax
import jax.numpy as jnp
from jax import lax
from jax.experimental import pallas as pl
from jax.experimental.pallas import tpu as pltpu
from jax.experimental.pallas import tpu_sc as plsc

N = 10000
E = 160000
DIN, DH, DOUT = 256, 512, 128
M_BLK = 1000
GRID_M = N // M_BLK
NCHUNK = DH // 128

SC_CORES = 2
SC_TILES = 16
EB = 128
E_PAD = 163840
R_ACC = 10240
S_IDX = 10016
ROWS_PER_TILE = R_ACC // SC_TILES
OW = 624
OW_LAST = N - (SC_TILES - 1) * OW


@functools.cache
def _vmesh():
    return plsc.VectorSubcoreMesh(core_axis_name="c", subcore_axis_name="s",
                                  num_cores=SC_CORES, num_subcores=SC_TILES)


@functools.cache
def _sc_params():
    cp = pltpu.CompilerParams()
    if "needs_layout_passes" in pltpu.CompilerParams.__dataclass_fields__:
        import dataclasses
        cp = dataclasses.replace(cp, needs_layout_passes=False)
    return cp


def _mm0_body(x_ref, w_ref, g_ref):
    g_ref[...] = jnp.dot(x_ref[...], w_ref[...],
                         preferred_element_type=jnp.float32)


def _tc_mm0(x, w0):
    return pl.pallas_call(
        _mm0_body,
        grid=(GRID_M,),
        in_specs=[
            pl.BlockSpec((M_BLK, DIN), lambda m: (m, 0)),
            pl.BlockSpec((DIN, DH), lambda m: (0, 0)),
        ],
        out_specs=pl.BlockSpec((M_BLK, DH), lambda m: (m, 0)),
        out_shape=jax.ShapeDtypeStruct((N, DH), jnp.float32),
    )(x, w0)


def _deg_terms(cnt_ref):
    cnt = cnt_ref[...]
    deg = cnt[0] + cnt[1] + 1.0
    dis = lax.rsqrt(deg)
    inv = 1.0 / deg
    return dis, inv


def _scale_body(g_ref, cnt_ref, gs_ref):
    dis, _ = _deg_terms(cnt_ref)
    gd = g_ref[...] * dis
    for c in range(NCHUNK):
        gs_ref[c] = gd[:, 128 * c:128 * (c + 1)]


def _tc_scale0(g0, cnt2):
    return pl.pallas_call(
        _scale_body,
        grid=(GRID_M,),
        in_specs=[
            pl.BlockSpec((M_BLK, DH), lambda m: (m, 0)),
            pl.BlockSpec((2, M_BLK, 1), lambda m: (0, m, 0)),
        ],
        out_specs=pl.BlockSpec((NCHUNK, M_BLK, 128), lambda m: (0, m, 0)),
        out_shape=jax.ShapeDtypeStruct((NCHUNK, N, 128), jnp.float32),
    )(g0, cnt2)


def _mid_body(dout, s_ref, gp_ref, cnt_ref, b_ref, w_ref, g_ref, gs_ref):
    dis, inv = _deg_terms(cnt_ref)
    h = jnp.maximum(dis * s_ref[...] + inv * gp_ref[...] + b_ref[...], 0.0)
    g = jnp.dot(h, w_ref[...], preferred_element_type=jnp.float32)
    g_ref[...] = g
    gd = g * dis
    nchunk = dout // 128
    for c in range(nchunk):
        gs_ref[c] = gd[:, 128 * c:128 * (c + 1)]


def _tc_mid(s, g_prev, cnt2, b, w):
    din, dout = w.shape
    nchunk = dout // 128
    return pl.pallas_call(
        functools.partial(_mid_body, dout),
        grid=(GRID_M,),
        in_specs=[
            pl.BlockSpec((M_BLK, din), lambda m: (m, 0)),
            pl.BlockSpec((M_BLK, din), lambda m: (m, 0)),
            pl.BlockSpec((2, M_BLK, 1), lambda m: (0, m, 0)),
            pl.BlockSpec((1, din), lambda m: (0, 0)),
            pl.BlockSpec((din, dout), lambda m: (0, 0)),
        ],
        out_specs=[
            pl.BlockSpec((M_BLK, dout), lambda m: (m, 0)),
            pl.BlockSpec((nchunk, M_BLK, 128), lambda m: (0, m, 0)),
        ],
        out_shape=[
            jax.ShapeDtypeStruct((N, dout), jnp.float32),
            jax.ShapeDtypeStruct((nchunk, N, 128), jnp.float32),
        ],
    )(s, g_prev, cnt2, b, w)


def _final_body(s_ref, g_ref, cnt_ref, b_ref, o_ref):
    dis, inv = _deg_terms(cnt_ref)
    s = s_ref[0] + s_ref[1]
    pre = dis * s + inv * g_ref[...] + b_ref[...]
    m = jnp.max(pre, axis=1, keepdims=True)
    y = pre - m
    lse = jnp.log(jnp.sum(jnp.exp(y), axis=1, keepdims=True))
    o_ref[...] = y - lse


def _tc_final(s3p, g3, cnt2, b3):
    return pl.pallas_call(
        _final_body,
        grid=(GRID_M,),
        in_specs=[
            pl.BlockSpec((2, M_BLK, DOUT), lambda m: (0, m, 0)),
            pl.BlockSpec((M_BLK, DOUT), lambda m: (m, 0)),
            pl.BlockSpec((2, M_BLK, 1), lambda m: (0, m, 0)),
            pl.BlockSpec((1, DOUT), lambda m: (0, 0)),
        ],
        out_specs=pl.BlockSpec((M_BLK, DOUT), lambda m: (m, 0)),
        out_shape=jax.ShapeDtypeStruct((N, DOUT), jnp.float32),
    )(s3p, g3, cnt2, b3)


def _prep_body(src_ref, out_ref):
    s = src_ref[...]
    for c in range(NCHUNK):
        out_ref[c] = s + c * N


def _tc_prep_src(srcp):
    pb = 2048
    rows = E_PAD // pb
    rb = 8
    return pl.pallas_call(
        _prep_body,
        grid=(rows // rb,),
        in_specs=[pl.BlockSpec((rb, pb), lambda m: (m, 0))],
        out_specs=pl.BlockSpec((NCHUNK, rb, pb), lambda m: (0, m, 0)),
        out_shape=jax.ShapeDtypeStruct((NCHUNK, rows, pb), jnp.int32),
    )(srcp.reshape(rows, pb)).reshape(NCHUNK * E_PAD)


def _zero_vmem_f32(ref, n):
    z = jnp.zeros((16,), jnp.float32)

    @pl.loop(0, n // 16)
    def _(i):
        ref[pl.ds(i * 16, 16)] = z


def _zero_rows(ref2d):
    z = jnp.zeros((16,), jnp.float32)

    @pl.loop(0, ref2d.shape[0])
    def _(i):
        for j in range(ref2d.shape[1] // 16):
            ref2d[i, pl.ds(16 * j, 16)] = z


def _sc_deg(dstp):
    eb_per_w = E_PAD // (SC_CORES * SC_TILES)

    @functools.partial(
        pl.kernel,
        out_type=jax.ShapeDtypeStruct((SC_CORES * N,), jnp.float32),
        mesh=_vmesh(),
        compiler_params=_sc_params(),
        scratch_types=[
            pltpu.VMEM((R_ACC,), jnp.float32),
            pltpu.VMEM((EB,), jnp.int32),
            pltpu.VMEM((SC_TILES, ROWS_PER_TILE), jnp.float32),
            pltpu.VMEM((ROWS_PER_TILE,), jnp.float32),
            pltpu.VMEM_SHARED((SC_TILES, R_ACC), jnp.float32),
        ],
    )
    def k(dst_hbm, cnt_hbm, deg_v, didx, part, outv, shared):
        core = lax.axis_index("c")
        s = lax.axis_index("s")
        wid = core * SC_TILES + s
        _zero_vmem_f32(deg_v, R_ACC)
        ones = jnp.ones((16,), jnp.float32)

        @pl.loop(0, eb_per_w // EB)
        def _(b):
            off = wid * eb_per_w + b * EB
            pltpu.sync_copy(dst_hbm.at[pl.ds(off, EB)], didx)
            for j in range(EB // 16):
                idx = didx[pl.ds(16 * j, 16)]
                plsc.addupdate_scatter(deg_v, [idx], ones)

        pltpu.sync_copy(deg_v, shared.at[s])
        plsc.subcore_barrier()
        pltpu.sync_copy(shared.at[:, pl.ds(s * ROWS_PER_TILE, ROWS_PER_TILE)],
                        part)

        @pl.loop(0, ROWS_PER_TILE // 16)
        def _(i):
            v = jnp.zeros((16,), jnp.float32)
            for r in range(SC_TILES):
                v = v + part[r, pl.ds(16 * i, 16)]
            outv[pl.ds(16 * i, 16)] = v

        @pl.when(s < SC_TILES - 1)
        def _():
            pltpu.sync_copy(outv,
                            cnt_hbm.at[pl.ds(core * N + s * ROWS_PER_TILE,
                                             ROWS_PER_TILE)])

        @pl.when(s == SC_TILES - 1)
        def _():
            last = N - (SC_TILES - 1) * ROWS_PER_TILE
            pltpu.sync_copy(outv.at[pl.ds(0, last)],
                            cnt_hbm.at[pl.ds(core * N + s * ROWS_PER_TILE,
                                             last)])

    return k(dstp)


def _edge_pass(gs_hbm, acc, src_hbm, dst_hbm, src_buf, dst_buf, rows0, rows1,
               gsem0, gsem1, src_row0, dst_row0, nbh):
    pltpu.sync_copy(src_hbm.at[pl.ds(src_row0, nbh)], src_buf)
    pltpu.sync_copy(dst_hbm.at[pl.ds(dst_row0, nbh)], dst_buf)
    _pipelined_edges(gs_hbm, acc, src_buf, dst_buf, rows0, rows1,
                     gsem0, gsem1, nbh)


def _pipelined_edges(gs_hbm, acc, src_all, dst_all, rows0, rows1,
                     gsem0, gsem1, nb):
    pltpu.async_copy(gs_hbm.at[src_all.at[0]], rows0, gsem0)
    pltpu.async_copy(gs_hbm.at[src_all.at[1]], rows1, gsem1)

    @pl.loop(0, nb // 2)
    def _(i):
        b0 = 2 * i
        pltpu.make_async_copy(gs_hbm.at[src_all.at[b0]], rows0, gsem0).wait()
        pltpu.sync_copy(rows0, acc.at[dst_all.at[b0]], add=True)

        @pl.when(i < nb // 2 - 1)
        def _():
            pltpu.async_copy(gs_hbm.at[src_all.at[b0 + 2]], rows0, gsem0)

        pltpu.make_async_copy(gs_hbm.at[src_all.at[b0 + 1]], rows1,
                              gsem1).wait()
        pltpu.sync_copy(rows1, acc.at[dst_all.at[b0 + 1]], add=True)

        @pl.when(i < nb // 2 - 1)
        def _():
            pltpu.async_copy(gs_hbm.at[src_all.at[b0 + 3]], rows1, gsem1)


def _sc_prop4(gs_flat, src4, dstp):
    eb_per_t = E_PAD // SC_TILES
    nb = eb_per_t // EB
    nbh = nb // 2

    @functools.partial(
        pl.kernel,
        out_type=jax.ShapeDtypeStruct((N, DH), jnp.float32),
        mesh=_vmesh(),
        compiler_params=_sc_params(),
        scratch_types=[
            pltpu.VMEM_SHARED((R_ACC, 128), jnp.float32),
            pltpu.VMEM((EB, 128), jnp.float32),
            pltpu.VMEM((EB, 128), jnp.float32),
            pltpu.VMEM((nbh, EB), jnp.int32),
            pltpu.VMEM((nbh, EB), jnp.int32),
            pltpu.SemaphoreType.DMA,
            pltpu.SemaphoreType.DMA,
        ],
    )
    def k(gs_hbm, src_hbm, dst_hbm, out_hbm, acc, rows0, rows1,
          src_all, dst_all, gsem0, gsem1):
        core = lax.axis_index("c")
        s = lax.axis_index("s")

        for cc in range(NCHUNK // SC_CORES):
            chunk = core * (NCHUNK // SC_CORES) + cc

            _zero_rows(rows0)
            for zi in range(ROWS_PER_TILE // EB):
                pltpu.sync_copy(
                    rows0, acc.at[pl.ds(s * ROWS_PER_TILE + zi * EB, EB)])
            plsc.subcore_barrier()

            for p in range(2):
                _edge_pass(gs_hbm, acc, src_hbm, dst_hbm, src_all, dst_all,
                           rows0, rows1, gsem0, gsem1,
                           chunk * (E_PAD // EB) + s * nb + p * nbh,
                           s * nb + p * nbh, nbh)

            plsc.subcore_barrier()

            @pl.when(s < SC_TILES - 1)
            def _():
                pltpu.sync_copy(
                    acc.at[pl.ds(s * OW, OW)],
                    out_hbm.at[pl.ds(s * OW, OW), pl.ds(chunk * 128, 128)])

            @pl.when(s == SC_TILES - 1)
            def _():
                pltpu.sync_copy(
                    acc.at[pl.ds((SC_TILES - 1) * OW, OW_LAST)],
                    out_hbm.at[pl.ds((SC_TILES - 1) * OW, OW_LAST),
                               pl.ds(chunk * 128, 128)])

            plsc.subcore_barrier()

    return k(gs_flat, src4, dstp)


def _sc_prop1(gs3, srcp, dstp):
    eb_per_t = E_PAD // (SC_CORES * SC_TILES)
    nb = eb_per_t // EB

    @functools.partial(
        pl.kernel,
        out_type=jax.ShapeDtypeStruct((SC_CORES, N, DOUT), jnp.float32),
        mesh=_vmesh(),
        compiler_params=_sc_params(),
        scratch_types=[
            pltpu.VMEM_SHARED((R_ACC, 128), jnp.float32),
            pltpu.VMEM((EB, 128), jnp.float32),
            pltpu.VMEM((EB, 128), jnp.float32),
            pltpu.VMEM((nb, EB), jnp.int32),
            pltpu.VMEM((nb, EB), jnp.int32),
            pltpu.SemaphoreType.DMA,
            pltpu.SemaphoreType.DMA,
        ],
    )
    def k(gs_hbm, src_hbm, dst_hbm, out_hbm, acc, rows0, rows1,
          src_all, dst_all, gsem0, gsem1):
        core = lax.axis_index("c")
        s = lax.axis_index("s")

        wrow = (core * SC_TILES + s) * nb

        _zero_rows(rows0)
        for zi in range(ROWS_PER_TILE // EB):
            pltpu.sync_copy(rows0,
                            acc.at[pl.ds(s * ROWS_PER_TILE + zi * EB, EB)])
        plsc.subcore_barrier()

        _edge_pass(gs_hbm, acc, src_hbm, dst_hbm, src_all, dst_all,
                   rows0, rows1, gsem0, gsem1, wrow, wrow, nb)

        plsc.subcore_barrier()

        @pl.when(s < SC_TILES - 1)
        def _():
            pltpu.sync_copy(acc.at[pl.ds(s * OW, OW)],
                            out_hbm.at[core, pl.ds(s * OW, OW), :])

        @pl.when(s == SC_TILES - 1)
        def _():
            pltpu.sync_copy(acc.at[pl.ds((SC_TILES - 1) * OW, OW_LAST)],
                            out_hbm.at[core, pl.ds((SC_TILES - 1) * OW,
                                                   OW_LAST), :])

    return k(gs3, srcp, dstp)


def kernel(x, edge_index, W0, b0, W1, b1, W2, b2, W3, b3):
    pad = E_PAD - E
    srcp = jnp.concatenate(
        [edge_index[0], jnp.zeros((pad,), edge_index.dtype)]).astype(jnp.int32)
    dstp = jnp.concatenate(
        [edge_index[1],
         jnp.full((pad,), S_IDX, edge_index.dtype)]).astype(jnp.int32)

    cnt2 = _sc_deg(dstp).reshape(2, N, 1)
    g0 = _tc_mm0(x, W0)
    src4 = _tc_prep_src(srcp).reshape(NCHUNK * E_PAD // EB, EB)
    src2d = srcp.reshape(E_PAD // EB, EB)
    dst2d = dstp.reshape(E_PAD // EB, EB)

    gs0 = _tc_scale0(g0, cnt2)
    s0 = _sc_prop4(gs0.reshape(NCHUNK * N, 128), src4, dst2d)
    g1, gs1 = _tc_mid(s0, g0, cnt2, b0.reshape(1, DH), W1)
    s1 = _sc_prop4(gs1.reshape(NCHUNK * N, 128), src4, dst2d)
    g2, gs2 = _tc_mid(s1, g1, cnt2, b1.reshape(1, DH), W2)
    s2 = _sc_prop4(gs2.reshape(NCHUNK * N, 128), src4, dst2d)
    g3, gs3 = _tc_mid(s2, g2, cnt2, b2.reshape(1, DH), W3)
    s3p = _sc_prop1(gs3.reshape(N, DOUT), src2d, dst2d)
    return _tc_final(s3p, g3, cnt2, b3.reshape(1, DOUT))
